# Initial kernel scaffold; baseline (speedup 1.0000x reference)
import jax
import jax.numpy as jnp
from jax.experimental import pallas as pl


def kernel(x, edge_index, W1_0, W1_1, b1, W2_0, W2_1, b2):
    raise NotImplementedError("write your pallas kernel here")


# speedup vs baseline: 13.4713x; 13.4713x over previous
import functools

import jax
import jax.numpy as jnp
from jax import lax
from jax.experimental import pallas as pl
from jax.experimental.pallas import tpu as pltpu
from jax.experimental.pallas import tpu_sc as plsc

N_NODES = 10000
N_EDGES = 160000
NC, NS = 2, 16
NW = NC * NS
EPW = N_EDGES // NW
CH = 120
NFULL = EPW // CH
TAIL = EPW - NFULL * CH
N_PAD = 10240
RPT = N_PAD // NS

def _mesh():
    return plsc.VectorSubcoreMesh(core_axis_name="c", subcore_axis_name="s")


FD = 16


@functools.lru_cache(maxsize=None)
def _make_deg_kernel():
    @functools.partial(
        pl.kernel,
        out_type=jax.ShapeDtypeStruct((NC, N_PAD, FD), jnp.float32),
        mesh=_mesh(),
        scratch_types=[
            pltpu.VMEM((CH,), jnp.int32),
            pltpu.VMEM((TAIL,), jnp.int32),
            pltpu.VMEM((CH, FD), jnp.float32),
            pltpu.VMEM((TAIL, FD), jnp.float32),
            pltpu.VMEM_SHARED((N_PAD, FD), jnp.float32),
        ],
        compiler_params=pltpu.CompilerParams(use_tc_tiling_on_sc=False),
    )
    def deg_kernel(src_hbm, ones_hbm, zeros_hbm, out_hbm,
                   idx_v, idxt_v, ones_v, onest_v, acc):
        c = lax.axis_index("c")
        s = lax.axis_index("s")
        w = s * NC + c
        pltpu.sync_copy(zeros_hbm.at[pl.ds(s * RPT, RPT)],
                        acc.at[pl.ds(s * RPT, RPT)])
        pltpu.sync_copy(ones_hbm, ones_v)
        pltpu.sync_copy(ones_hbm.at[pl.ds(0, TAIL)], onest_v)
        plsc.subcore_barrier()
        base = pl.multiple_of(w * EPW, 8)

        def step(i, carry):
            off = pl.multiple_of(base + i * CH, 8)
            pltpu.sync_copy(src_hbm.at[pl.ds(off, CH)], idx_v)
            pltpu.sync_copy(ones_v, acc.at[idx_v], add=True)
            return carry

        lax.fori_loop(0, NFULL, step, 0)
        toff = pl.multiple_of(base + NFULL * CH, 8)
        pltpu.sync_copy(src_hbm.at[pl.ds(toff, TAIL)], idxt_v)
        pltpu.sync_copy(onest_v, acc.at[idxt_v], add=True)
        plsc.subcore_barrier()
        pltpu.sync_copy(acc.at[pl.ds(s * RPT, RPT)],
                        out_hbm.at[c].at[pl.ds(s * RPT, RPT)])

    return deg_kernel


@functools.lru_cache(maxsize=None)
def _make_scatter_kernel(F):
    @functools.partial(
        pl.kernel,
        out_type=jax.ShapeDtypeStruct((NC, N_PAD, F), jnp.float32),
        mesh=_mesh(),
        scratch_types=[
            pltpu.VMEM((CH,), jnp.int32),
            pltpu.VMEM((CH,), jnp.int32),
            pltpu.VMEM((TAIL,), jnp.int32),
            pltpu.VMEM((TAIL,), jnp.int32),
            pltpu.VMEM((CH, F), jnp.float32),
            pltpu.VMEM((TAIL, F), jnp.float32),
            pltpu.VMEM_SHARED((N_PAD, F), jnp.float32),
            pltpu.SemaphoreType.DMA,
        ],
        compiler_params=pltpu.CompilerParams(use_tc_tiling_on_sc=False),
    )
    def scatter_kernel(u_hbm, src_hbm, dst_hbm, zeros_hbm, out_hbm,
                       sidx, didx, sidx_t, didx_t, rows, rows_t, acc, sem):
        c = lax.axis_index("c")
        s = lax.axis_index("s")
        w = s * NC + c
        pltpu.sync_copy(zeros_hbm.at[pl.ds(s * RPT, RPT)],
                        acc.at[pl.ds(s * RPT, RPT)])
        plsc.subcore_barrier()
        base = pl.multiple_of(w * EPW, 8)

        def step(i, carry):
            off = pl.multiple_of(base + i * CH, 8)
            pltpu.sync_copy(src_hbm.at[pl.ds(off, CH)], sidx)
            pltpu.sync_copy(dst_hbm.at[pl.ds(off, CH)], didx)
            pltpu.async_copy(u_hbm.at[sidx], rows, sem).wait()
            pltpu.sync_copy(rows, acc.at[didx], add=True)
            return carry

        lax.fori_loop(0, NFULL, step, 0)
        toff = pl.multiple_of(base + NFULL * CH, 8)
        pltpu.sync_copy(src_hbm.at[pl.ds(toff, TAIL)], sidx_t)
        pltpu.sync_copy(dst_hbm.at[pl.ds(toff, TAIL)], didx_t)
        pltpu.async_copy(u_hbm.at[sidx_t], rows_t, sem).wait()
        pltpu.sync_copy(rows_t, acc.at[didx_t], add=True)
        plsc.subcore_barrier()
        pltpu.sync_copy(acc.at[pl.ds(s * RPT, RPT)],
                        out_hbm.at[c].at[pl.ds(s * RPT, RPT)])

    return scatter_kernel


def _t1_body(x_ref, w0_ref, w1_ref, degp_ref, y0_ref, u1_ref, dis_ref):
    deg = degp_ref[0, :N_NODES, :1] + degp_ref[1, :N_NODES, :1]
    dis = jnp.where(deg > 0, lax.rsqrt(jnp.maximum(deg, 1e-12)), 0.0)
    dis_ref[...] = dis
    xv = x_ref[...]
    y0_ref[...] = jnp.dot(xv, w0_ref[...], preferred_element_type=jnp.float32)
    u1_ref[...] = dis * jnp.dot(xv, w1_ref[...],
                                preferred_element_type=jnp.float32)


def _t2_body(y0_ref, accp_ref, dis_ref, b1_ref, w20_ref, w21_ref,
             z0_ref, u2_ref):
    dis = dis_ref[...]
    tx = dis * (accp_ref[0, :N_NODES] + accp_ref[1, :N_NODES])
    h = jnp.maximum(y0_ref[...] - tx + b1_ref[...], 0.0)
    z0_ref[...] = jnp.dot(h, w20_ref[...], preferred_element_type=jnp.float32)
    u2_ref[...] = dis * jnp.dot(h, w21_ref[...],
                                preferred_element_type=jnp.float32)


def _t3_body(z0_ref, accp_ref, dis_ref, b2_ref, out_ref):
    tx = dis_ref[...] * (accp_ref[0, :N_NODES] + accp_ref[1, :N_NODES])[:, :40]
    o = z0_ref[...] - tx + b2_ref[...]
    m = jnp.max(o, axis=1, keepdims=True)
    e = jnp.exp(o - m)
    out_ref[...] = o - m - jnp.log(jnp.sum(e, axis=1, keepdims=True))


def kernel(x, edge_index, W1_0, W1_1, b1, W2_0, W2_1, b2):
    src = edge_index[0]
    dst = edge_index[1]
    ones_ch = jnp.ones((CH, FD), jnp.float32)
    zeros1 = jnp.zeros((N_PAD, FD), jnp.float32)
    zeros32 = jnp.zeros((N_PAD, 32), jnp.float32)
    zeros48 = jnp.zeros((N_PAD, 48), jnp.float32)
    w21p = jnp.pad(W2_1, ((0, 0), (0, 8)))

    degp = _make_deg_kernel()(src, ones_ch, zeros1)

    y0, u1, dis = pl.pallas_call(
        _t1_body,
        out_shape=[
            jax.ShapeDtypeStruct((N_NODES, 32), jnp.float32),
            jax.ShapeDtypeStruct((N_NODES, 32), jnp.float32),
            jax.ShapeDtypeStruct((N_NODES, 1), jnp.float32),
        ],
    )(x, W1_0, W1_1, degp)

    acc1 = _make_scatter_kernel(32)(u1, src, dst, zeros32)

    z0, u2 = pl.pallas_call(
        _t2_body,
        out_shape=[
            jax.ShapeDtypeStruct((N_NODES, 40), jnp.float32),
            jax.ShapeDtypeStruct((N_NODES, 48), jnp.float32),
        ],
    )(y0, acc1, dis, b1, W2_0, w21p)

    acc2 = _make_scatter_kernel(48)(u2, src, dst, zeros48)

    out = pl.pallas_call(
        _t3_body,
        out_shape=jax.ShapeDtypeStruct((N_NODES, 40), jnp.float32),
    )(z0, acc2, dis, b2)
    return out


# speedup vs baseline: 14.8968x; 1.1058x over previous
import functools

import jax
import jax.numpy as jnp
from jax import lax
from jax.experimental import pallas as pl
from jax.experimental.pallas import tpu as pltpu
from jax.experimental.pallas import tpu_sc as plsc

N_NODES = 10000
N_EDGES = 160000
NC, NS = 2, 16
NW = NC * NS
CH = 128
CPW = 40
E_PAD = NW * CPW * CH
NB = 8
N_PAD = 10240
RPT = N_PAD // NS
PAD_IDX = N_NODES

def _mesh():
    return plsc.VectorSubcoreMesh(core_axis_name="c", subcore_axis_name="s")


FD = 16


@functools.lru_cache(maxsize=None)
def _make_deg_kernel():
    @functools.partial(
        pl.kernel,
        out_type=jax.ShapeDtypeStruct((NC, N_PAD, FD), jnp.float32),
        mesh=_mesh(),
        scratch_types=[
            pltpu.VMEM((CPW, CH), jnp.int32),
            pltpu.VMEM((CH, FD), jnp.float32),
            pltpu.VMEM_SHARED((N_PAD, FD), jnp.float32),
            pltpu.SemaphoreType.DMA,
        ],
        compiler_params=pltpu.CompilerParams(use_tc_tiling_on_sc=False),
    )
    def deg_kernel(src_hbm, ones_hbm, zeros_hbm, out_hbm,
                   idx_v, ones_v, acc, sem):
        c = lax.axis_index("c")
        s = lax.axis_index("s")
        w = s * NC + c
        pltpu.sync_copy(zeros_hbm.at[pl.ds(s * RPT, RPT)],
                        acc.at[pl.ds(s * RPT, RPT)])
        pltpu.sync_copy(ones_hbm, ones_v)
        rbase = pl.multiple_of(w * CPW, 8)
        pltpu.sync_copy(src_hbm.at[pl.ds(rbase, CPW)], idx_v)
        plsc.subcore_barrier()

        def rnd(g, carry):
            ds = [pltpu.async_copy(ones_v, acc.at[idx_v.at[g * NB + b]], sem,
                                   add=True)
                  for b in range(NB)]
            for d in ds:
                d.wait()
            return carry

        lax.fori_loop(0, CPW // NB, rnd, 0)
        plsc.subcore_barrier()
        pltpu.sync_copy(acc.at[pl.ds(s * RPT, RPT)],
                        out_hbm.at[c].at[pl.ds(s * RPT, RPT)])

    return deg_kernel


@functools.lru_cache(maxsize=None)
def _make_scatter_kernel(F):
    @functools.partial(
        pl.kernel,
        out_type=jax.ShapeDtypeStruct((NC, N_PAD, F), jnp.float32),
        mesh=_mesh(),
        scratch_types=(
            [pltpu.VMEM((CPW, CH), jnp.int32)] * 2
            + [pltpu.VMEM((CH, F), jnp.float32)] * NB
            + [pltpu.VMEM_SHARED((N_PAD, F), jnp.float32)]
            + [pltpu.SemaphoreType.DMA] * (2 * NB)
        ),
        compiler_params=pltpu.CompilerParams(use_tc_tiling_on_sc=False),
    )
    def scatter_kernel(u_hbm, src_hbm, dst_hbm, zeros_hbm, out_hbm,
                       sidx, didx, *bufs):
        rows = bufs[:NB]
        acc = bufs[NB]
        gsem = bufs[NB + 1:2 * NB + 1]
        ssem = bufs[2 * NB + 1:]
        c = lax.axis_index("c")
        s = lax.axis_index("s")
        w = s * NC + c
        pltpu.sync_copy(zeros_hbm.at[pl.ds(s * RPT, RPT)],
                        acc.at[pl.ds(s * RPT, RPT)])
        rbase = pl.multiple_of(w * CPW, 8)
        pltpu.sync_copy(src_hbm.at[pl.ds(rbase, CPW)], sidx)
        pltpu.sync_copy(dst_hbm.at[pl.ds(rbase, CPW)], didx)
        plsc.subcore_barrier()

        def rnd(g, carry):
            gds = [pltpu.async_copy(u_hbm.at[sidx.at[g * NB + b]], rows[b],
                                    gsem[b])
                   for b in range(NB)]
            sds = []
            for b in range(NB):
                gds[b].wait()
                sds.append(pltpu.async_copy(rows[b],
                                            acc.at[didx.at[g * NB + b]],
                                            ssem[b], add=True))
            for d in sds:
                d.wait()
            return carry

        lax.fori_loop(0, CPW // NB, rnd, 0)
        plsc.subcore_barrier()
        pltpu.sync_copy(acc.at[pl.ds(s * RPT, RPT)],
                        out_hbm.at[c].at[pl.ds(s * RPT, RPT)])

    return scatter_kernel


def _t1_body(x_ref, w0_ref, w1_ref, degp_ref, y0_ref, u1_ref, dis_ref):
    deg = degp_ref[0, :N_NODES, :1] + degp_ref[1, :N_NODES, :1]
    dis = jnp.where(deg > 0, lax.rsqrt(jnp.maximum(deg, 1e-12)), 0.0)
    dis_ref[...] = dis
    xv = x_ref[...]
    y0_ref[...] = jnp.dot(xv, w0_ref[...], preferred_element_type=jnp.float32)
    u1_ref[...] = dis * jnp.dot(xv, w1_ref[...],
                                preferred_element_type=jnp.float32)


def _t2_body(y0_ref, accp_ref, dis_ref, b1_ref, w20_ref, w21_ref,
             z0_ref, u2_ref):
    dis = dis_ref[...]
    tx = dis * (accp_ref[0, :N_NODES] + accp_ref[1, :N_NODES])
    h = jnp.maximum(y0_ref[...] - tx + b1_ref[...], 0.0)
    z0_ref[...] = jnp.dot(h, w20_ref[...], preferred_element_type=jnp.float32)
    u2_ref[...] = dis * jnp.dot(h, w21_ref[...],
                                preferred_element_type=jnp.float32)


def _t3_body(z0_ref, accp_ref, dis_ref, b2_ref, out_ref):
    tx = dis_ref[...] * (accp_ref[0, :N_NODES] + accp_ref[1, :N_NODES])[:, :40]
    o = z0_ref[...] - tx + b2_ref[...]
    m = jnp.max(o, axis=1, keepdims=True)
    e = jnp.exp(o - m)
    out_ref[...] = o - m - jnp.log(jnp.sum(e, axis=1, keepdims=True))


def kernel(x, edge_index, W1_0, W1_1, b1, W2_0, W2_1, b2):
    src = edge_index[0]
    dst = edge_index[1]
    npad = E_PAD - N_EDGES
    src_gat = jnp.concatenate(
        [src, jnp.zeros((npad,), jnp.int32)]).reshape(-1, CH)
    src_deg = jnp.concatenate(
        [src, jnp.full((npad,), PAD_IDX, jnp.int32)]).reshape(-1, CH)
    dst_pad = jnp.concatenate(
        [dst, jnp.full((npad,), PAD_IDX, jnp.int32)]).reshape(-1, CH)
    ones_ch = jnp.ones((CH, FD), jnp.float32)
    zeros1 = jnp.zeros((N_PAD, FD), jnp.float32)
    zeros32 = jnp.zeros((N_PAD, 32), jnp.float32)
    zeros48 = jnp.zeros((N_PAD, 48), jnp.float32)
    w21p = jnp.pad(W2_1, ((0, 0), (0, 8)))

    degp = _make_deg_kernel()(src_deg, ones_ch, zeros1)

    y0, u1, dis = pl.pallas_call(
        _t1_body,
        out_shape=[
            jax.ShapeDtypeStruct((N_NODES, 32), jnp.float32),
            jax.ShapeDtypeStruct((N_NODES, 32), jnp.float32),
            jax.ShapeDtypeStruct((N_NODES, 1), jnp.float32),
        ],
    )(x, W1_0, W1_1, degp)

    acc1 = _make_scatter_kernel(32)(u1, src_gat, dst_pad, zeros32)

    z0, u2 = pl.pallas_call(
        _t2_body,
        out_shape=[
            jax.ShapeDtypeStruct((N_NODES, 40), jnp.float32),
            jax.ShapeDtypeStruct((N_NODES, 48), jnp.float32),
        ],
    )(y0, acc1, dis, b1, W2_0, w21p)

    acc2 = _make_scatter_kernel(48)(u2, src_gat, dst_pad, zeros48)

    out = pl.pallas_call(
        _t3_body,
        out_shape=jax.ShapeDtypeStruct((N_NODES, 40), jnp.float32),
    )(z0, acc2, dis, b2)
    return out


# speedup vs baseline: 21.9712x; 1.4749x over previous
import functools

import jax
import jax.numpy as jnp
from jax import lax
from jax.experimental import pallas as pl
from jax.experimental.pallas import tpu as pltpu
from jax.experimental.pallas import tpu_sc as plsc

N_NODES = 10000
N_EDGES = 160000
NC, NS = 2, 16
NW = NC * NS
CH = 128
CPW = 40
E_PAD = NW * CPW * CH
NB = 8
N_PAD = 10240
RPT = N_PAD // NS
PAD_IDX = N_NODES

def _mesh():
    return plsc.VectorSubcoreMesh(core_axis_name="c", subcore_axis_name="s")


FD = 16


@functools.lru_cache(maxsize=None)
def _make_deg_kernel():
    @functools.partial(
        pl.kernel,
        out_type=jax.ShapeDtypeStruct((NC, N_PAD, FD), jnp.float32),
        mesh=_mesh(),
        scratch_types=[
            pltpu.VMEM((CPW, CH), jnp.int32),
            pltpu.VMEM((CH, FD), jnp.float32),
            pltpu.VMEM_SHARED((N_PAD, FD), jnp.float32),
            pltpu.SemaphoreType.DMA,
        ],
        compiler_params=pltpu.CompilerParams(use_tc_tiling_on_sc=False),
    )
    def deg_kernel(src_hbm, ones_hbm, zeros_hbm, out_hbm,
                   idx_v, ones_v, acc, sem):
        c = lax.axis_index("c")
        s = lax.axis_index("s")
        w = s * NC + c
        pltpu.sync_copy(zeros_hbm.at[pl.ds(s * RPT, RPT)],
                        acc.at[pl.ds(s * RPT, RPT)])
        pltpu.sync_copy(ones_hbm, ones_v)
        rbase = pl.multiple_of(w * CPW, 8)
        pltpu.sync_copy(src_hbm.at[pl.ds(rbase, CPW)], idx_v)
        plsc.subcore_barrier()

        def rnd(g, carry):
            ds = [pltpu.async_copy(ones_v, acc.at[idx_v.at[g * NB + b]], sem,
                                   add=True)
                  for b in range(NB)]
            for d in ds:
                d.wait()
            return carry

        lax.fori_loop(0, CPW // NB, rnd, 0)
        plsc.subcore_barrier()
        pltpu.sync_copy(acc.at[pl.ds(s * RPT, RPT)],
                        out_hbm.at[c].at[pl.ds(s * RPT, RPT)])

    return deg_kernel


@functools.lru_cache(maxsize=None)
def _make_scatter_kernel(F):
    @functools.partial(
        pl.kernel,
        out_type=jax.ShapeDtypeStruct((NC, N_PAD, F), jnp.float32),
        mesh=_mesh(),
        scratch_types=(
            [pltpu.VMEM((CPW, CH), jnp.int32)] * 2
            + [pltpu.VMEM((CH, F), jnp.float32)] * NB
            + [pltpu.VMEM_SHARED((N_PAD, F), jnp.float32)] * 2
            + [pltpu.SemaphoreType.DMA] * (2 * NB)
        ),
        compiler_params=pltpu.CompilerParams(use_tc_tiling_on_sc=False),
    )
    def scatter_kernel(u_hbm, src_hbm, dst_hbm, zeros_hbm, out_hbm,
                       sidx, didx, *bufs):
        rows = bufs[:NB]
        acc = bufs[NB]
        u_s = bufs[NB + 1]
        gsem = bufs[NB + 2:2 * NB + 2]
        ssem = bufs[2 * NB + 2:]
        c = lax.axis_index("c")
        s = lax.axis_index("s")
        w = s * NC + c
        pltpu.sync_copy(zeros_hbm.at[pl.ds(s * RPT, RPT)],
                        acc.at[pl.ds(s * RPT, RPT)])
        pltpu.sync_copy(u_hbm.at[pl.ds(s * RPT, RPT)],
                        u_s.at[pl.ds(s * RPT, RPT)])
        rbase = pl.multiple_of(w * CPW, 8)
        pltpu.sync_copy(src_hbm.at[pl.ds(rbase, CPW)], sidx)
        pltpu.sync_copy(dst_hbm.at[pl.ds(rbase, CPW)], didx)
        plsc.subcore_barrier()

        def rnd(g, carry):
            gds = [pltpu.async_copy(u_s.at[sidx.at[g * NB + b]], rows[b],
                                    gsem[b])
                   for b in range(NB)]
            sds = []
            for b in range(NB):
                gds[b].wait()
                sds.append(pltpu.async_copy(rows[b],
                                            acc.at[didx.at[g * NB + b]],
                                            ssem[b], add=True))
            for d in sds:
                d.wait()
            return carry

        lax.fori_loop(0, CPW // NB, rnd, 0)
        plsc.subcore_barrier()
        pltpu.sync_copy(acc.at[pl.ds(s * RPT, RPT)],
                        out_hbm.at[c].at[pl.ds(s * RPT, RPT)])

    return scatter_kernel


def _t1_body(x_ref, w0_ref, w1_ref, degp_ref, y0_ref, u1_ref, dis_ref):
    deg = degp_ref[0, :N_NODES, :1] + degp_ref[1, :N_NODES, :1]
    dis = jnp.where(deg > 0, lax.rsqrt(jnp.maximum(deg, 1e-12)), 0.0)
    dis_ref[...] = dis
    xv = x_ref[...]
    y0_ref[...] = jnp.dot(xv, w0_ref[...], preferred_element_type=jnp.float32)
    u1_ref[...] = dis * jnp.dot(xv, w1_ref[...],
                                preferred_element_type=jnp.float32)


def _t2_body(y0_ref, accp_ref, dis_ref, b1_ref, w20_ref, w21_ref,
             z0_ref, u2_ref):
    dis = dis_ref[...]
    tx = dis * (accp_ref[0, :N_NODES] + accp_ref[1, :N_NODES])
    h = jnp.maximum(y0_ref[...] - tx + b1_ref[...], 0.0)
    z0_ref[...] = jnp.dot(h, w20_ref[...], preferred_element_type=jnp.float32)
    u2_ref[...] = dis * jnp.dot(h, w21_ref[...],
                                preferred_element_type=jnp.float32)


def _t3_body(z0_ref, accp_ref, dis_ref, b2_ref, out_ref):
    tx = dis_ref[...] * (accp_ref[0, :N_NODES] + accp_ref[1, :N_NODES])[:, :40]
    o = z0_ref[...] - tx + b2_ref[...]
    m = jnp.max(o, axis=1, keepdims=True)
    e = jnp.exp(o - m)
    out_ref[...] = o - m - jnp.log(jnp.sum(e, axis=1, keepdims=True))


def kernel(x, edge_index, W1_0, W1_1, b1, W2_0, W2_1, b2):
    src = edge_index[0]
    dst = edge_index[1]
    npad = E_PAD - N_EDGES
    src_gat = jnp.concatenate(
        [src, jnp.zeros((npad,), jnp.int32)]).reshape(-1, CH)
    src_deg = jnp.concatenate(
        [src, jnp.full((npad,), PAD_IDX, jnp.int32)]).reshape(-1, CH)
    dst_pad = jnp.concatenate(
        [dst, jnp.full((npad,), PAD_IDX, jnp.int32)]).reshape(-1, CH)
    ones_ch = jnp.ones((CH, FD), jnp.float32)
    zeros1 = jnp.zeros((N_PAD, FD), jnp.float32)
    zeros32 = jnp.zeros((N_PAD, 32), jnp.float32)
    zeros48 = jnp.zeros((N_PAD, 48), jnp.float32)
    w21p = jnp.pad(W2_1, ((0, 0), (0, 8)))

    degp = _make_deg_kernel()(src_deg, ones_ch, zeros1)

    y0, u1, dis = pl.pallas_call(
        _t1_body,
        out_shape=[
            jax.ShapeDtypeStruct((N_NODES, 32), jnp.float32),
            jax.ShapeDtypeStruct((N_NODES, 32), jnp.float32),
            jax.ShapeDtypeStruct((N_NODES, 1), jnp.float32),
        ],
    )(x, W1_0, W1_1, degp)

    u1p = jnp.pad(u1, ((0, N_PAD - N_NODES), (0, 0)))
    acc1 = _make_scatter_kernel(32)(u1p, src_gat, dst_pad, zeros32)

    z0, u2 = pl.pallas_call(
        _t2_body,
        out_shape=[
            jax.ShapeDtypeStruct((N_NODES, 40), jnp.float32),
            jax.ShapeDtypeStruct((N_NODES, 48), jnp.float32),
        ],
    )(y0, acc1, dis, b1, W2_0, w21p)

    u2p = jnp.pad(u2, ((0, N_PAD - N_NODES), (0, 0)))
    acc2 = _make_scatter_kernel(48)(u2p, src_gat, dst_pad, zeros48)

    out = pl.pallas_call(
        _t3_body,
        out_shape=jax.ShapeDtypeStruct((N_NODES, 40), jnp.float32),
    )(z0, acc2, dis, b2)
    return out


# speedup vs baseline: 22.7342x; 1.0347x over previous
import functools

import jax
import jax.numpy as jnp
from jax import lax
from jax.experimental import pallas as pl
from jax.experimental.pallas import tpu as pltpu
from jax.experimental.pallas import tpu_sc as plsc

N_NODES = 10000
N_EDGES = 160000
NC, NS = 2, 16
NW = NC * NS
CH = 128
CPW = 40
E_PAD = NW * CPW * CH
NB = 8
N_PAD = 10240
RPT = N_PAD // NS
PAD_IDX = N_NODES

def _mesh():
    return plsc.VectorSubcoreMesh(core_axis_name="c", subcore_axis_name="s")


FD = 16


@functools.lru_cache(maxsize=None)
def _make_deg_kernel():
    @functools.partial(
        pl.kernel,
        out_type=jax.ShapeDtypeStruct((NC, N_PAD, FD), jnp.float32),
        mesh=_mesh(),
        scratch_types=[
            pltpu.VMEM((CPW, CH), jnp.int32),
            pltpu.VMEM((CH, FD), jnp.float32),
            pltpu.VMEM_SHARED((N_PAD, FD), jnp.float32),
            pltpu.SemaphoreType.DMA,
        ],
        compiler_params=pltpu.CompilerParams(use_tc_tiling_on_sc=False),
    )
    def deg_kernel(src_hbm, ones_hbm, zeros_hbm, out_hbm,
                   idx_v, ones_v, acc, sem):
        c = lax.axis_index("c")
        s = lax.axis_index("s")
        w = s * NC + c
        pltpu.sync_copy(zeros_hbm.at[pl.ds(s * RPT, RPT)],
                        acc.at[pl.ds(s * RPT, RPT)])
        pltpu.sync_copy(ones_hbm, ones_v)
        rbase = pl.multiple_of(w * CPW, 8)
        pltpu.sync_copy(src_hbm.at[pl.ds(rbase, CPW)], idx_v)
        plsc.subcore_barrier()

        def rnd(g, carry):
            ds = [pltpu.async_copy(ones_v, acc.at[idx_v.at[g * NB + b]], sem,
                                   add=True)
                  for b in range(NB)]
            for d in ds:
                d.wait()
            return carry

        lax.fori_loop(0, CPW // NB, rnd, 0)
        plsc.subcore_barrier()
        pltpu.sync_copy(acc.at[pl.ds(s * RPT, RPT)],
                        out_hbm.at[c].at[pl.ds(s * RPT, RPT)])

    return deg_kernel


@functools.lru_cache(maxsize=None)
def _make_scatter_kernel(F):
    @functools.partial(
        pl.kernel,
        out_type=jax.ShapeDtypeStruct((NC, N_PAD, F), jnp.float32),
        mesh=_mesh(),
        scratch_types=(
            [pltpu.VMEM((CPW, CH), jnp.int32)] * 2
            + [pltpu.VMEM((CH, F), jnp.float32)] * NB
            + [pltpu.VMEM_SHARED((N_PAD, F), jnp.float32)] * 2
            + [pltpu.SemaphoreType.DMA] * (2 * NB)
        ),
        compiler_params=pltpu.CompilerParams(use_tc_tiling_on_sc=False),
    )
    def scatter_kernel(u_hbm, src_hbm, dst_hbm, zeros_hbm, out_hbm,
                       sidx, didx, *bufs):
        rows = bufs[:NB]
        acc = bufs[NB]
        u_s = bufs[NB + 1]
        gsem = bufs[NB + 2:2 * NB + 2]
        ssem = bufs[2 * NB + 2:]
        c = lax.axis_index("c")
        s = lax.axis_index("s")
        w = s * NC + c
        pltpu.sync_copy(zeros_hbm.at[pl.ds(s * RPT, RPT)],
                        acc.at[pl.ds(s * RPT, RPT)])
        pltpu.sync_copy(u_hbm.at[pl.ds(s * RPT, RPT)],
                        u_s.at[pl.ds(s * RPT, RPT)])
        rbase = pl.multiple_of(w * CPW, 8)
        pltpu.sync_copy(src_hbm.at[pl.ds(rbase, CPW)], sidx)
        pltpu.sync_copy(dst_hbm.at[pl.ds(rbase, CPW)], didx)
        plsc.subcore_barrier()

        def rnd(g, carry):
            gds = [pltpu.async_copy(u_s.at[sidx.at[g * NB + b]], rows[b],
                                    gsem[b])
                   for b in range(NB)]
            sds = []
            for b in range(NB):
                gds[b].wait()
                sds.append(pltpu.async_copy(rows[b],
                                            acc.at[didx.at[g * NB + b]],
                                            ssem[b], add=True))
            for d in sds:
                d.wait()
            return carry

        lax.fori_loop(0, CPW // NB, rnd, 0)
        plsc.subcore_barrier()
        pltpu.sync_copy(acc.at[pl.ds(s * RPT, RPT)],
                        out_hbm.at[c].at[pl.ds(s * RPT, RPT)])

    return scatter_kernel


def _t1_body(x_ref, w0_ref, w1_ref, degp_ref, y0_ref, u1_ref, dis_ref):
    deg = degp_ref[0, :N_NODES, :1] + degp_ref[1, :N_NODES, :1]
    dis = jnp.where(deg > 0, lax.rsqrt(jnp.maximum(deg, 1e-12)), 0.0)
    dis_ref[...] = dis
    xv = x_ref[...]
    y0_ref[...] = jnp.dot(xv, w0_ref[...], preferred_element_type=jnp.float32)
    u1_ref[...] = dis * jnp.dot(xv, w1_ref[...],
                                preferred_element_type=jnp.float32)


def _t2_body(y0_ref, accp_ref, dis_ref, b1_ref, w20_ref, w21_ref,
             z0_ref, u2_ref):
    dis = dis_ref[...]
    tx = dis * (accp_ref[0, :N_NODES] + accp_ref[1, :N_NODES])
    h = jnp.maximum(y0_ref[...] - tx + b1_ref[...], 0.0)
    z0_ref[...] = jnp.dot(h, w20_ref[...], preferred_element_type=jnp.float32)
    u2_ref[...] = dis * jnp.dot(h, w21_ref[...],
                                preferred_element_type=jnp.float32)


def _t3_body(z0_ref, accp_ref, dis_ref, b2_ref, out_ref):
    tx = dis_ref[...] * (accp_ref[0, :N_NODES] + accp_ref[1, :N_NODES])[:, :40]
    o = z0_ref[...] - tx + b2_ref[...]
    m = jnp.max(o, axis=1, keepdims=True)
    e = jnp.exp(o - m)
    out_ref[...] = o - m - jnp.log(jnp.sum(e, axis=1, keepdims=True))


def kernel(x, edge_index, W1_0, W1_1, b1, W2_0, W2_1, b2):
    ei2 = jnp.pad(edge_index, ((0, 0), (0, E_PAD - N_EDGES)),
                  constant_values=PAD_IDX)
    src_pad = ei2[0].reshape(-1, CH)
    dst_pad = ei2[1].reshape(-1, CH)
    ones_ch = jnp.ones((CH, FD), jnp.float32)
    zeros1 = jnp.zeros((N_PAD, FD), jnp.float32)
    zeros32 = jnp.zeros((N_PAD, 32), jnp.float32)
    zeros48 = jnp.zeros((N_PAD, 48), jnp.float32)
    w21p = jnp.pad(W2_1, ((0, 0), (0, 8)))

    degp = _make_deg_kernel()(src_pad, ones_ch, zeros1)

    y0, u1, dis = pl.pallas_call(
        _t1_body,
        out_shape=[
            jax.ShapeDtypeStruct((N_NODES, 32), jnp.float32),
            jax.ShapeDtypeStruct((N_NODES, 32), jnp.float32),
            jax.ShapeDtypeStruct((N_NODES, 1), jnp.float32),
        ],
    )(x, W1_0, W1_1, degp)

    u1p = jnp.pad(u1, ((0, N_PAD - N_NODES), (0, 0)))
    acc1 = _make_scatter_kernel(32)(u1p, src_pad, dst_pad, zeros32)

    z0, u2 = pl.pallas_call(
        _t2_body,
        out_shape=[
            jax.ShapeDtypeStruct((N_NODES, 40), jnp.float32),
            jax.ShapeDtypeStruct((N_NODES, 48), jnp.float32),
        ],
    )(y0, acc1, dis, b1, W2_0, w21p)

    u2p = jnp.pad(u2, ((0, N_PAD - N_NODES), (0, 0)))
    acc2 = _make_scatter_kernel(48)(u2p, src_pad, dst_pad, zeros48)

    out = pl.pallas_call(
        _t3_body,
        out_shape=jax.ShapeDtypeStruct((N_NODES, 40), jnp.float32),
    )(z0, acc2, dis, b2)
    return out


# speedup vs baseline: 23.3191x; 1.0257x over previous
import functools

import jax
import jax.numpy as jnp
from jax import lax
from jax.experimental import pallas as pl
from jax.experimental.pallas import tpu as pltpu
from jax.experimental.pallas import tpu_sc as plsc

N_NODES = 10000
N_EDGES = 160000
NC, NS = 2, 16
NW = NC * NS
CH = 128
CPW = 40
E_PAD = NW * CPW * CH
NB = 8
N_PAD = 10240
RPT = N_PAD // NS
PAD_IDX = N_NODES

def _mesh():
    return plsc.VectorSubcoreMesh(core_axis_name="c", subcore_axis_name="s")


FD = 8


@functools.lru_cache(maxsize=None)
def _make_deg_kernel():
    @functools.partial(
        pl.kernel,
        out_type=jax.ShapeDtypeStruct((NC, N_PAD, FD), jnp.float32),
        mesh=_mesh(),
        scratch_types=[
            pltpu.VMEM((CPW, CH), jnp.int32),
            pltpu.VMEM((CH, FD), jnp.float32),
            pltpu.VMEM_SHARED((N_PAD, FD), jnp.float32),
            pltpu.SemaphoreType.DMA,
        ],
        compiler_params=pltpu.CompilerParams(use_tc_tiling_on_sc=False),
    )
    def deg_kernel(src_hbm, ones_hbm, zeros_hbm, out_hbm,
                   idx_v, ones_v, acc, sem):
        c = lax.axis_index("c")
        s = lax.axis_index("s")
        w = s * NC + c
        pltpu.sync_copy(zeros_hbm.at[pl.ds(s * RPT, RPT)],
                        acc.at[pl.ds(s * RPT, RPT)])
        pltpu.sync_copy(ones_hbm, ones_v)
        rbase = pl.multiple_of(w * CPW, 8)
        pltpu.sync_copy(src_hbm.at[pl.ds(rbase, CPW)], idx_v)
        plsc.subcore_barrier()

        def rnd(g, carry):
            ds = [pltpu.async_copy(ones_v, acc.at[idx_v.at[g * NB + b]], sem,
                                   add=True)
                  for b in range(NB)]
            for d in ds:
                d.wait()
            return carry

        lax.fori_loop(0, CPW // NB, rnd, 0)
        plsc.subcore_barrier()
        pltpu.sync_copy(acc.at[pl.ds(s * RPT, RPT)],
                        out_hbm.at[c].at[pl.ds(s * RPT, RPT)])

    return deg_kernel


@functools.lru_cache(maxsize=None)
def _make_scatter_kernel(F):
    @functools.partial(
        pl.kernel,
        out_type=jax.ShapeDtypeStruct((NC, N_PAD, F), jnp.float32),
        mesh=_mesh(),
        scratch_types=(
            [pltpu.VMEM((CPW, CH), jnp.int32)] * 2
            + [pltpu.VMEM((CH, F), jnp.float32)] * NB
            + [pltpu.VMEM_SHARED((N_PAD, F), jnp.float32)] * 2
            + [pltpu.SemaphoreType.DMA] * (2 * NB)
        ),
        compiler_params=pltpu.CompilerParams(use_tc_tiling_on_sc=False),
    )
    def scatter_kernel(u_hbm, src_hbm, dst_hbm, zeros_hbm, out_hbm,
                       sidx, didx, *bufs):
        rows = bufs[:NB]
        acc = bufs[NB]
        u_s = bufs[NB + 1]
        gsem = bufs[NB + 2:2 * NB + 2]
        ssem = bufs[2 * NB + 2:]
        c = lax.axis_index("c")
        s = lax.axis_index("s")
        w = s * NC + c
        pltpu.sync_copy(zeros_hbm.at[pl.ds(s * RPT, RPT)],
                        acc.at[pl.ds(s * RPT, RPT)])
        pltpu.sync_copy(u_hbm.at[pl.ds(s * RPT, RPT)],
                        u_s.at[pl.ds(s * RPT, RPT)])
        rbase = pl.multiple_of(w * CPW, 8)
        pltpu.sync_copy(src_hbm.at[pl.ds(rbase, CPW)], sidx)
        pltpu.sync_copy(dst_hbm.at[pl.ds(rbase, CPW)], didx)
        plsc.subcore_barrier()

        def rnd(g, carry):
            gds = [pltpu.async_copy(u_s.at[sidx.at[g * NB + b]], rows[b],
                                    gsem[b])
                   for b in range(NB)]
            sds = []
            for b in range(NB):
                gds[b].wait()
                sds.append(pltpu.async_copy(rows[b],
                                            acc.at[didx.at[g * NB + b]],
                                            ssem[b], add=True))
            for d in sds:
                d.wait()
            return carry

        lax.fori_loop(0, CPW // NB, rnd, 0)
        plsc.subcore_barrier()
        pltpu.sync_copy(acc.at[pl.ds(s * RPT, RPT)],
                        out_hbm.at[c].at[pl.ds(s * RPT, RPT)])

    return scatter_kernel


def _t1_body(x_ref, w0_ref, w1_ref, degp_ref, y0_ref, u1_ref, dis_ref):
    deg = degp_ref[0, :N_NODES, :1] + degp_ref[1, :N_NODES, :1]
    dis = jnp.where(deg > 0, lax.rsqrt(jnp.maximum(deg, 1e-12)), 0.0)
    dis_ref[...] = dis
    xv = x_ref[...]
    y0_ref[...] = jnp.dot(xv, w0_ref[...], preferred_element_type=jnp.float32)
    u1_ref[...] = dis * jnp.dot(xv, w1_ref[...],
                                preferred_element_type=jnp.float32)


def _t2_body(y0_ref, accp_ref, dis_ref, b1_ref, w20_ref, w21_ref,
             z0_ref, u2_ref):
    dis = dis_ref[...]
    tx = dis * (accp_ref[0, :N_NODES] + accp_ref[1, :N_NODES])
    h = jnp.maximum(y0_ref[...] - tx + b1_ref[...], 0.0)
    z0_ref[...] = jnp.dot(h, w20_ref[...], preferred_element_type=jnp.float32)
    u2_ref[...] = dis * jnp.dot(h, w21_ref[...],
                                preferred_element_type=jnp.float32)


def _t3_body(z0_ref, accp_ref, dis_ref, b2_ref, out_ref):
    tx = dis_ref[...] * (accp_ref[0, :N_NODES] + accp_ref[1, :N_NODES])
    o = z0_ref[...] - tx + b2_ref[...]
    m = jnp.max(o, axis=1, keepdims=True)
    e = jnp.exp(o - m)
    out_ref[...] = o - m - jnp.log(jnp.sum(e, axis=1, keepdims=True))


def kernel(x, edge_index, W1_0, W1_1, b1, W2_0, W2_1, b2):
    ei2 = jnp.pad(edge_index, ((0, 0), (0, E_PAD - N_EDGES)),
                  constant_values=PAD_IDX)
    src_pad = ei2[0].reshape(-1, CH)
    dst_pad = ei2[1].reshape(-1, CH)
    ones_ch = jnp.ones((CH, FD), jnp.float32)
    zeros1 = jnp.zeros((N_PAD, FD), jnp.float32)
    zeros32 = jnp.zeros((N_PAD, 32), jnp.float32)
    zeros40 = jnp.zeros((N_PAD, 40), jnp.float32)

    degp = _make_deg_kernel()(src_pad, ones_ch, zeros1)

    y0, u1, dis = pl.pallas_call(
        _t1_body,
        out_shape=[
            jax.ShapeDtypeStruct((N_NODES, 32), jnp.float32),
            jax.ShapeDtypeStruct((N_NODES, 32), jnp.float32),
            jax.ShapeDtypeStruct((N_NODES, 1), jnp.float32),
        ],
    )(x, W1_0, W1_1, degp)

    u1p = jnp.pad(u1, ((0, N_PAD - N_NODES), (0, 0)))
    acc1 = _make_scatter_kernel(32)(u1p, src_pad, dst_pad, zeros32)

    z0, u2 = pl.pallas_call(
        _t2_body,
        out_shape=[
            jax.ShapeDtypeStruct((N_NODES, 40), jnp.float32),
            jax.ShapeDtypeStruct((N_NODES, 40), jnp.float32),
        ],
    )(y0, acc1, dis, b1, W2_0, W2_1)

    u2p = jnp.pad(u2, ((0, N_PAD - N_NODES), (0, 0)))
    acc2 = _make_scatter_kernel(40)(u2p, src_pad, dst_pad, zeros40)

    out = pl.pallas_call(
        _t3_body,
        out_shape=jax.ShapeDtypeStruct((N_NODES, 40), jnp.float32),
    )(z0, acc2, dis, b2)
    return out
